# baseline (speedup 1.0000x reference)
import functools

import jax
import jax.numpy as jnp
from jax import lax
from jax.experimental import pallas as pl
from jax.experimental.pallas import tpu as pltpu
from jax.experimental.pallas import tpu_sc as plsc

_NUM_GROUPS = 16
_FEAT = 256
_FH = 64
_FW = 64
_HW = _FH * _FW
_ROWS = _NUM_GROUPS * _HW

_NC = 2
_NS = 16
_NW = _NC * _NS
_CHUNK = 112
_NBUF = 4
_LAG = 2


def _gather(table, grp, xs, ys, n):
    n_chunks = -(-n // _CHUNK)
    last_chunk = n_chunks - 1
    last_base = n - _CHUNK
    assert last_base % 8 == 0 and _CHUNK % 8 == 0
    chunks_per_w = -(-n_chunks // _NW)
    assert chunks_per_w % _NBUF == 0 and chunks_per_w >= 2 * _NBUF
    mesh = plsc.VectorSubcoreMesh(core_axis_name="c", subcore_axis_name="s")

    @functools.partial(
        pl.kernel,
        mesh=mesh,
        compiler_params=pltpu.CompilerParams(needs_layout_passes=False),
        out_type=jax.ShapeDtypeStruct((n, _FEAT), jnp.float32),
        scratch_types=(
            [pltpu.VMEM((_CHUNK,), jnp.int32) for _ in range(_NBUF)]
            + [pltpu.VMEM((_CHUNK,), jnp.int32) for _ in range(_NBUF)]
            + [pltpu.VMEM((_CHUNK,), jnp.int32) for _ in range(_NBUF)]
            + [pltpu.VMEM((_CHUNK,), jnp.int32) for _ in range(_NBUF)]
            + [pltpu.VMEM((_CHUNK, _FEAT), jnp.float32) for _ in range(_NBUF)]
            + [pltpu.SemaphoreType.DMA for _ in range(3 * _NBUF)]
        ),
    )
    def k(table_hbm, grp_hbm, xs_hbm, ys_hbm, out_hbm, *scratch):
        grp_b = scratch[0:_NBUF]
        xs_b = scratch[_NBUF:2 * _NBUF]
        ys_b = scratch[2 * _NBUF:3 * _NBUF]
        idx_b = scratch[3 * _NBUF:4 * _NBUF]
        rows_b = scratch[4 * _NBUF:5 * _NBUF]
        ssem_b = scratch[5 * _NBUF:6 * _NBUF]
        gsem_b = scratch[6 * _NBUF:7 * _NBUF]
        wsem_b = scratch[7 * _NBUF:8 * _NBUF]

        wid = lax.axis_index("s") * _NC + lax.axis_index("c")

        def chunk_base(j):
            t = jnp.minimum(wid + j * _NW, last_chunk)
            return jnp.minimum(t * _CHUNK, last_base)

        def start_slabs(j, b):
            base = chunk_base(j)
            pltpu.async_copy(grp_hbm.at[pl.ds(base, _CHUNK)], grp_b[b], ssem_b[b])
            pltpu.async_copy(xs_hbm.at[pl.ds(base, _CHUNK)], xs_b[b], ssem_b[b])
            pltpu.async_copy(ys_hbm.at[pl.ds(base, _CHUNK)], ys_b[b], ssem_b[b])

        def wait_slabs(b):
            for ref in (grp_b[b], xs_b[b], ys_b[b]):
                pltpu.make_async_copy(
                    grp_hbm.at[pl.ds(0, _CHUNK)], ref, ssem_b[b]).wait()

        def compute_idx(b):
            for i in range(_CHUNK // 16):
                sl = pl.ds(i * 16, 16)
                g = grp_b[b][sl]
                x = xs_b[b][sl]
                y = ys_b[b][sl]
                idx_b[b][sl] = g * _HW + y * _FW + x

        def wait_write(b):
            pltpu.make_async_copy(
                rows_b[b], out_hbm.at[pl.ds(0, _CHUNK)], wsem_b[b]).wait()

        def wait_gather(b):
            pltpu.make_async_copy(
                table_hbm.at[idx_b[b]], rows_b[b], gsem_b[b]).wait()

        def drain_write(j, b):
            wait_gather(b)
            pltpu.async_copy(
                rows_b[b], out_hbm.at[pl.ds(chunk_base(j), _CHUNK)], wsem_b[b])

        def chunk(j, b, ring_warm, have_prev):
            wait_slabs(b)
            compute_idx(b)
            start_slabs(j + _NBUF, b)
            if ring_warm:
                wait_write(b)
            pltpu.async_copy(table_hbm.at[idx_b[b]], rows_b[b], gsem_b[b])
            if have_prev:
                drain_write(j - _LAG, (b - _LAG) % _NBUF)

        for b in range(_NBUF):
            start_slabs(jnp.int32(b), b)
        for b in range(_NBUF):
            chunk(jnp.int32(b), b, False, b >= _LAG)

        def body(jj, carry):
            j = jj * _NBUF
            for b in range(_NBUF):
                chunk(j + b, b, True, True)
            return carry

        lax.fori_loop(1, chunks_per_w // _NBUF, body, 0)

        for i in range(_LAG):
            j = chunks_per_w - _LAG + i
            drain_write(jnp.int32(j), j % _NBUF)
        for b in range(_NBUF):
            wait_write(b)
            wait_slabs(b)

    return k(table, grp, xs, ys)


def kernel(feat_map, grp_ids, grid_ids):
    n = grp_ids.shape[0]
    table = jnp.transpose(feat_map, (0, 2, 3, 1)).reshape(_ROWS, _FEAT)
    grp = grp_ids.astype(jnp.int32)
    xs = grid_ids[:, 0].astype(jnp.int32)
    ys = grid_ids[:, 1].astype(jnp.int32)
    return _gather(table, grp, xs, ys, n)


# speedup vs baseline: 1.0044x; 1.0044x over previous
import functools

import jax
import jax.numpy as jnp
from jax import lax
from jax.experimental import pallas as pl
from jax.experimental.pallas import tpu as pltpu
from jax.experimental.pallas import tpu_sc as plsc

_NUM_GROUPS = 16
_FEAT = 256
_FH = 64
_FW = 64
_HW = _FH * _FW
_ROWS = _NUM_GROUPS * _HW

_NC = 2
_NS = 16
_NW = _NC * _NS
_CHUNK = 128
_NBUF = 3
_LAG = 2


def _gather(table, grp, xs, ys, n):
    n_chunks = -(-n // _CHUNK)
    last_chunk = n_chunks - 1
    last_base = n - _CHUNK
    assert last_base % 8 == 0 and _CHUNK % 8 == 0
    chunks_per_w = -(-n_chunks // _NW)
    assert chunks_per_w % _NBUF == 1 and chunks_per_w >= 2 * _NBUF
    mesh = plsc.VectorSubcoreMesh(core_axis_name="c", subcore_axis_name="s")

    @functools.partial(
        pl.kernel,
        mesh=mesh,
        compiler_params=pltpu.CompilerParams(needs_layout_passes=False),
        out_type=jax.ShapeDtypeStruct((n, _FEAT), jnp.float32),
        scratch_types=(
            [pltpu.VMEM((_CHUNK,), jnp.int32) for _ in range(_NBUF)]
            + [pltpu.VMEM((_CHUNK,), jnp.int32) for _ in range(_NBUF)]
            + [pltpu.VMEM((_CHUNK,), jnp.int32) for _ in range(_NBUF)]
            + [pltpu.VMEM((_CHUNK,), jnp.int32) for _ in range(_NBUF)]
            + [pltpu.VMEM((_CHUNK, _FEAT), jnp.float32) for _ in range(_NBUF)]
            + [pltpu.SemaphoreType.DMA for _ in range(3 * _NBUF)]
        ),
    )
    def k(table_hbm, grp_hbm, xs_hbm, ys_hbm, out_hbm, *scratch):
        grp_b = scratch[0:_NBUF]
        xs_b = scratch[_NBUF:2 * _NBUF]
        ys_b = scratch[2 * _NBUF:3 * _NBUF]
        idx_b = scratch[3 * _NBUF:4 * _NBUF]
        rows_b = scratch[4 * _NBUF:5 * _NBUF]
        ssem_b = scratch[5 * _NBUF:6 * _NBUF]
        gsem_b = scratch[6 * _NBUF:7 * _NBUF]
        wsem_b = scratch[7 * _NBUF:8 * _NBUF]

        wid = lax.axis_index("s") * _NC + lax.axis_index("c")

        def chunk_base(j):
            t = jnp.minimum(wid + j * _NW, last_chunk)
            return jnp.minimum(t * _CHUNK, last_base)

        def start_slabs(j, b):
            base = chunk_base(j)
            pltpu.async_copy(grp_hbm.at[pl.ds(base, _CHUNK)], grp_b[b], ssem_b[b])
            pltpu.async_copy(xs_hbm.at[pl.ds(base, _CHUNK)], xs_b[b], ssem_b[b])
            pltpu.async_copy(ys_hbm.at[pl.ds(base, _CHUNK)], ys_b[b], ssem_b[b])

        def wait_slabs(b):
            for ref in (grp_b[b], xs_b[b], ys_b[b]):
                pltpu.make_async_copy(
                    grp_hbm.at[pl.ds(0, _CHUNK)], ref, ssem_b[b]).wait()

        def compute_idx(b):
            for i in range(_CHUNK // 16):
                sl = pl.ds(i * 16, 16)
                g = grp_b[b][sl]
                x = xs_b[b][sl]
                y = ys_b[b][sl]
                idx_b[b][sl] = g * _HW + y * _FW + x

        def wait_write(b):
            pltpu.make_async_copy(
                rows_b[b], out_hbm.at[pl.ds(0, _CHUNK)], wsem_b[b]).wait()

        def wait_gather(b):
            pltpu.make_async_copy(
                table_hbm.at[idx_b[b]], rows_b[b], gsem_b[b]).wait()

        def drain_write(j, b):
            wait_gather(b)
            pltpu.async_copy(
                rows_b[b], out_hbm.at[pl.ds(chunk_base(j), _CHUNK)], wsem_b[b])

        def chunk(j, b, ring_warm, have_prev):
            wait_slabs(b)
            compute_idx(b)
            start_slabs(j + _NBUF, b)
            if ring_warm:
                wait_write(b)
            pltpu.async_copy(table_hbm.at[idx_b[b]], rows_b[b], gsem_b[b])
            if have_prev:
                drain_write(j - _LAG, (b - _LAG) % _NBUF)

        for b in range(_NBUF):
            start_slabs(jnp.int32(b), b)
        for b in range(_NBUF):
            chunk(jnp.int32(b), b, False, b >= _LAG)

        def body(jj, carry):
            j = jj * _NBUF
            for b in range(_NBUF):
                chunk(j + b, b, True, True)
            return carry

        lax.fori_loop(1, chunks_per_w // _NBUF, body, 0)

        last = chunks_per_w - 1
        chunk(jnp.int32(last), last % _NBUF, True, True)
        for i in range(_LAG):
            j = chunks_per_w - _LAG + i
            drain_write(jnp.int32(j), j % _NBUF)
        for b in range(_NBUF):
            wait_write(b)
            wait_slabs(b)

    return k(table, grp, xs, ys)


def kernel(feat_map, grp_ids, grid_ids):
    n = grp_ids.shape[0]
    table = jnp.transpose(feat_map, (0, 2, 3, 1)).reshape(_ROWS, _FEAT)
    grp = grp_ids.astype(jnp.int32)
    xs = grid_ids[:, 0].astype(jnp.int32)
    ys = grid_ids[:, 1].astype(jnp.int32)
    return _gather(table, grp, xs, ys, n)


# speedup vs baseline: 1.0121x; 1.0077x over previous
import functools

import jax
import jax.numpy as jnp
from jax import lax
from jax.experimental import pallas as pl
from jax.experimental.pallas import tpu as pltpu
from jax.experimental.pallas import tpu_sc as plsc

_NUM_GROUPS = 16
_FEAT = 256
_FH = 64
_FW = 64
_HW = _FH * _FW
_ROWS = _NUM_GROUPS * _HW

_NC = 2
_NS = 16
_NW = _NC * _NS
_CHUNK = 128


def _gather(table, grp, xs, ys, n):
    n_chunks = -(-n // _CHUNK)
    last_chunk = n_chunks - 1
    last_base = n - _CHUNK
    chunks_per_w = -(-n_chunks // _NW)
    assert chunks_per_w % 2 == 1 and chunks_per_w >= 5
    mesh = plsc.VectorSubcoreMesh(core_axis_name="c", subcore_axis_name="s")

    @functools.partial(
        pl.kernel,
        mesh=mesh,
        compiler_params=pltpu.CompilerParams(needs_layout_passes=False),
        out_type=jax.ShapeDtypeStruct((n, _FEAT), jnp.float32),
        scratch_types=(
            [pltpu.VMEM((_CHUNK,), jnp.int32) for _ in range(2)]
            + [pltpu.VMEM((_CHUNK,), jnp.int32) for _ in range(2)]
            + [pltpu.VMEM((_CHUNK,), jnp.int32) for _ in range(2)]
            + [pltpu.VMEM((_CHUNK,), jnp.int32) for _ in range(2)]
            + [pltpu.VMEM((_CHUNK, _FEAT), jnp.float32) for _ in range(2)]
            + [pltpu.SemaphoreType.DMA for _ in range(6)]
        ),
    )
    def k(table_hbm, grp_hbm, xs_hbm, ys_hbm, out_hbm, *scratch):
        grp_b = scratch[0:2]
        xs_b = scratch[2:4]
        ys_b = scratch[4:6]
        idx_b = scratch[6:8]
        rows_b = scratch[8:10]
        ssem_b = scratch[10:12]
        gsem_b = scratch[12:14]
        wsem_b = scratch[14:16]

        wid = lax.axis_index("s") * _NC + lax.axis_index("c")

        def chunk_base(j):
            t = jnp.minimum(wid + j * _NW, last_chunk)
            return jnp.minimum(t * _CHUNK, last_base)

        def start_slabs(j, b):
            base = chunk_base(j)
            pltpu.async_copy(grp_hbm.at[pl.ds(base, _CHUNK)], grp_b[b], ssem_b[b])
            pltpu.async_copy(xs_hbm.at[pl.ds(base, _CHUNK)], xs_b[b], ssem_b[b])
            pltpu.async_copy(ys_hbm.at[pl.ds(base, _CHUNK)], ys_b[b], ssem_b[b])

        def wait_slabs(b):
            for ref in (grp_b[b], xs_b[b], ys_b[b]):
                pltpu.make_async_copy(
                    grp_hbm.at[pl.ds(0, _CHUNK)], ref, ssem_b[b]).wait()

        def compute_idx(b):
            for i in range(_CHUNK // 16):
                sl = pl.ds(i * 16, 16)
                g = grp_b[b][sl]
                x = xs_b[b][sl]
                y = ys_b[b][sl]
                idx_b[b][sl] = g * _HW + y * _FW + x

        def wait_write(b):
            pltpu.make_async_copy(
                rows_b[b], out_hbm.at[pl.ds(0, _CHUNK)], wsem_b[b]).wait()

        def wait_gather(b):
            pltpu.make_async_copy(
                table_hbm.at[idx_b[b]], rows_b[b], gsem_b[b]).wait()

        def chunk(j, b, ring_warm, have_prev):
            wait_slabs(b)
            compute_idx(b)
            start_slabs(j + 2, b)
            if ring_warm:
                wait_write(b)
            pltpu.async_copy(table_hbm.at[idx_b[b]], rows_b[b], gsem_b[b])
            if have_prev:
                pb = 1 - b
                wait_gather(pb)
                pltpu.async_copy(
                    rows_b[pb], out_hbm.at[pl.ds(chunk_base(j - 1), _CHUNK)],
                    wsem_b[pb])

        start_slabs(jnp.int32(0), 0)
        start_slabs(jnp.int32(1), 1)
        chunk(jnp.int32(0), 0, False, False)
        chunk(jnp.int32(1), 1, False, True)

        def body(jj, carry):
            j = jj * 2
            chunk(j, 0, True, True)
            chunk(j + 1, 1, True, True)
            return carry

        lax.fori_loop(1, chunks_per_w // 2, body, 0)

        chunk(jnp.int32(chunks_per_w - 1), 0, True, True)
        lb = 0
        wait_gather(lb)
        pltpu.async_copy(
            rows_b[lb],
            out_hbm.at[pl.ds(chunk_base(jnp.int32(chunks_per_w - 1)), _CHUNK)],
            wsem_b[lb])
        for b in range(2):
            wait_write(b)
            wait_slabs(b)

    return k(table, grp, xs, ys)


def kernel(feat_map, grp_ids, grid_ids):
    n = grp_ids.shape[0]
    table = jnp.transpose(feat_map, (0, 2, 3, 1)).reshape(_ROWS, _FEAT)
    grp = grp_ids.astype(jnp.int32)
    xs = grid_ids[:, 0].astype(jnp.int32)
    ys = grid_ids[:, 1].astype(jnp.int32)
    return _gather(table, grp, xs, ys, n)
